# baseline (speedup 1.0000x reference)
import jax
import jax.numpy as jnp
from jax import lax
from jax.experimental import pallas as pl
from jax.experimental.pallas import tpu as pltpu
from jax.experimental.pallas import tpu_sc as plsc

N = 10000
E = 320000
D = 128
O = 64

NC = 2
NS = 16
NW = NC * NS
EPW = E // NW
CC = 16
CB = 32
NBUF = 4
PREF = 2
PAD = NBUF * CB
NCHC = EPW // CC
NP = 10112
ROWS_PER_TILE = NP // NS

BN = 2000


def _stage_a_body(x_ref, wmsg_ref, wself_ref, wsrc_ref, wdst_ref, brel_ref,
                  xm_ref, xs_ref, ab_ref):
    xb = x_ref[...]
    xm_ref[...] = jnp.dot(xb, wmsg_ref[...], preferred_element_type=jnp.float32)
    xs_ref[...] = jnp.dot(xb, wself_ref[...], preferred_element_type=jnp.float32)
    ls = (jnp.dot(xb, wsrc_ref[...], preferred_element_type=jnp.float32)
          + brel_ref[...])
    ld = jnp.dot(xb, wdst_ref[...], preferred_element_type=jnp.float32)
    a = ls[:, 1:2] - ls[:, 0:1]
    b = ld[:, 1:2] - ld[:, 0:1]
    ai = lax.bitcast_convert_type(a, jnp.int32) + 0x8000
    bi = lax.bitcast_convert_type(b, jnp.int32) + 0x8000
    bits = jnp.bitwise_or(jnp.bitwise_and(bi, jnp.int32(-65536)),
                          lax.shift_right_logical(ai, 16))
    ab_ref[...] = lax.bitcast_convert_type(bits, jnp.float32)


def _stage_a(x, w_msg, w_self, w_rel_src, w_rel_dst, brel2):
    return pl.pallas_call(
        _stage_a_body,
        grid=(N // BN,),
        in_specs=[
            pl.BlockSpec((BN, D), lambda i: (i, 0)),
            pl.BlockSpec((D, D), lambda i: (0, 0)),
            pl.BlockSpec((D, D), lambda i: (0, 0)),
            pl.BlockSpec((D, 2), lambda i: (0, 0)),
            pl.BlockSpec((D, 2), lambda i: (0, 0)),
            pl.BlockSpec((1, 2), lambda i: (0, 0)),
        ],
        out_specs=[
            pl.BlockSpec((BN, D), lambda i: (i, 0)),
            pl.BlockSpec((BN, D), lambda i: (i, 0)),
            pl.BlockSpec((BN, 1), lambda i: (i, 0)),
        ],
        out_shape=[
            jax.ShapeDtypeStruct((N, D), jnp.float32),
            jax.ShapeDtypeStruct((N, D), jnp.float32),
            jax.ShapeDtypeStruct((N, 1), jnp.float32),
        ],
    )(x, w_msg, w_self, w_rel_src, w_rel_dst, brel2)


def _stage_b_body(xs_ref, agg_ref, wout_ref, bout_ref, out_ref):
    acc = xs_ref[...] + agg_ref[0] + agg_ref[1]
    h = jnp.maximum(acc, 0.0)
    out_ref[...] = (jnp.dot(h, wout_ref[...], preferred_element_type=jnp.float32)
                    + bout_ref[...])


def _stage_b(xs, agg2, w_out, bout2):
    return pl.pallas_call(
        _stage_b_body,
        grid=(N // BN,),
        in_specs=[
            pl.BlockSpec((BN, D), lambda i: (i, 0)),
            pl.BlockSpec((NC, BN, D), lambda i: (0, i, 0)),
            pl.BlockSpec((D, O), lambda i: (0, 0)),
            pl.BlockSpec((1, O), lambda i: (0, 0)),
        ],
        out_specs=pl.BlockSpec((BN, O), lambda i: (i, 0)),
        out_shape=jax.ShapeDtypeStruct((N, O), jnp.float32),
    )(xs, agg2, w_out, bout2)


def _sc_edge_body(xm_hbm, ab_hbm, edge_hbm, out_hbm,
                  src_v, dst_v, ab_v, rows_v, sidx_v, agg_sh, gsem, ssem):
    cid = lax.axis_index("c")
    sid = lax.axis_index("s")
    wid = cid * NS + sid
    ebase = wid * EPW

    pltpu.sync_copy(edge_hbm.at[pl.ds(ebase, EPW)], src_v.at[pl.ds(0, EPW)])
    pltpu.sync_copy(edge_hbm.at[pl.ds(E + ebase, EPW)], dst_v.at[pl.ds(0, EPW)])
    pltpu.sync_copy(ab_hbm, ab_v)

    zeros16 = jnp.zeros((16,), jnp.float32)
    for r in range(8):
        for c in range(D // 16):
            rows_v[0, r, pl.ds(c * 16, 16)] = zeros16
    rbase = sid * ROWS_PER_TILE

    def zero_loop(i, carry):
        pltpu.sync_copy(rows_v.at[0, pl.ds(0, 8)],
                        agg_sh.at[pl.ds(rbase + i * 8, 8)])
        return carry

    lax.fori_loop(0, ROWS_PER_TILE // 8, zero_loop, 0)

    def compact(g, off):
        sv = src_v[pl.ds(g * CC, CC)]
        dv = dst_v[pl.ds(g * CC, CC)]
        keep = sv <= dv
        plsc.store_compressed(src_v.at[pl.ds(off, CC)], sv, mask=keep)
        plsc.store_compressed(dst_v.at[pl.ds(off, CC)], dv, mask=keep)
        return off + plsc.all_reduce_population_count(keep)[0]

    off = lax.fori_loop(0, NCHC, compact, 0)
    for k in range(PAD // CC):
        src_v[pl.ds(off + k * CC, CC)] = jnp.full((CC,), 1, jnp.int32)
        dst_v[pl.ds(off + k * CC, CC)] = jnp.zeros((CC,), jnp.int32)
    nch2 = lax.max((off + CB - 1) // CB, NBUF)
    plsc.subcore_barrier()

    def gather_start(g, b):
        pltpu.async_copy(xm_hbm.at[src_v.at[pl.ds(g * CB, CB)]],
                         rows_v.at[b], gsem.at[b])

    def gather_wait(g, b):
        pltpu.make_async_copy(xm_hbm.at[src_v.at[pl.ds(g * CB, CB)]],
                              rows_v.at[b], gsem.at[b]).wait()

    def scatter_start(b):
        pltpu.async_copy(rows_v.at[b], agg_sh.at[sidx_v.at[b]], ssem.at[b],
                         add=True)

    def scatter_wait(b):
        pltpu.make_async_copy(rows_v.at[b], agg_sh.at[sidx_v.at[b]],
                              ssem.at[b]).wait()

    for b in range(PREF):
        gather_start(b, b)

    himask = jnp.full((16,), -65536, jnp.int32)

    def slot(g, carry):
        b = lax.rem(g, NBUF)
        gather_wait(g, b)
        for q in range(CB // 16):
            sv = src_v[pl.ds(g * CB + q * 16, 16)]
            dv = dst_v[pl.ds(g * CB + q * 16, 16)]
            g1 = plsc.bitcast(plsc.load_gather(ab_v, [sv]), jnp.int32)
            g2 = plsc.bitcast(plsc.load_gather(ab_v, [dv]), jnp.int32)
            av = plsc.bitcast(lax.shift_left(g1, 16), jnp.float32)
            bv = plsc.bitcast(g2 & himask, jnp.float32)
            p = 1.0 / (1.0 + jnp.exp(-(av + bv)))
            w = jnp.where(sv <= dv, p, 0.0)
            sidx_v[b, pl.ds(q * 16, 16)] = dv
            for j in range(16):
                s = w[j]
                r = q * 16 + j
                for c in range(D // 16):
                    rows_v[b, r, pl.ds(c * 16, 16)] = (
                        rows_v[b, r, pl.ds(c * 16, 16)] * s)
        scatter_start(b)
        gn = g + PREF
        bn = lax.rem(gn, NBUF)

        @pl.when(jnp.logical_and(gn >= NBUF, gn < nch2))
        def _():
            scatter_wait(bn)

        @pl.when(gn < nch2)
        def _():
            gather_start(gn, bn)

        return carry

    lax.fori_loop(0, nch2, slot, 0)
    for b in range(NBUF):
        scatter_wait(b)
    plsc.subcore_barrier()

    pltpu.sync_copy(agg_sh.at[pl.ds(rbase, ROWS_PER_TILE)],
                    out_hbm.at[cid, pl.ds(rbase, ROWS_PER_TILE)])


def _sc_edge(xm, ab_packed, edge_index):
    mesh = plsc.VectorSubcoreMesh(core_axis_name="c", subcore_axis_name="s")
    f = pl.kernel(
        _sc_edge_body,
        out_type=jax.ShapeDtypeStruct((NC, NP, D), jnp.float32),
        mesh=mesh,
        compiler_params=pltpu.CompilerParams(needs_layout_passes=False),
        scratch_types=[
            pltpu.VMEM((EPW + PAD,), jnp.int32),
            pltpu.VMEM((EPW + PAD,), jnp.int32),
            pltpu.VMEM((N,), jnp.float32),
            pltpu.VMEM((NBUF, CB, D), jnp.float32),
            pltpu.VMEM((NBUF, CB), jnp.int32),
            pltpu.VMEM_SHARED((NP, D), jnp.float32),
            pltpu.SemaphoreType.DMA((NBUF,)),
            pltpu.SemaphoreType.DMA((NBUF,)),
        ],
    )
    return f(xm, ab_packed, edge_index)


def kernel(x, edge_index, W_rel_src, W_rel_dst, b_rel, W_msg, W_self, W_out,
           b_out):
    xm, xs, ab = _stage_a(x, W_msg, W_self, W_rel_src, W_rel_dst,
                          b_rel.reshape(1, 2))
    agg2 = _sc_edge(xm, ab.reshape(-1), edge_index.reshape(-1))
    return _stage_b(xs, agg2, W_out, b_out.reshape(1, O))


# speedup vs baseline: 1.1719x; 1.1719x over previous
import jax
import jax.numpy as jnp
from jax import lax
from jax.experimental import pallas as pl
from jax.experimental.pallas import tpu as pltpu
from jax.experimental.pallas import tpu_sc as plsc

N = 10000
E = 320000
D = 128
O = 64

NC = 2
NS = 16
NW = NC * NS
EPW = E // NW
CC = 16
CB = 32
NBUF = 4
PREF = 3
PAD = NBUF * CB
NCHC = EPW // CC
NP = 10112
ROWS_PER_TILE = NP // NS

BN = 2000


def _stage_a_body(x_ref, wmsg_ref, wself_ref, wsrc_ref, wdst_ref, brel_ref,
                  xm_ref, xs_ref, ab_ref):
    xb = x_ref[...]
    xm_ref[...] = jnp.dot(xb, wmsg_ref[...], preferred_element_type=jnp.float32)
    xs_ref[...] = jnp.dot(xb, wself_ref[...], preferred_element_type=jnp.float32)
    ls = (jnp.dot(xb, wsrc_ref[...], preferred_element_type=jnp.float32)
          + brel_ref[...])
    ld = jnp.dot(xb, wdst_ref[...], preferred_element_type=jnp.float32)
    a = ls[:, 1:2] - ls[:, 0:1]
    b = ld[:, 1:2] - ld[:, 0:1]
    ai = lax.bitcast_convert_type(a, jnp.int32) + 0x8000
    bi = lax.bitcast_convert_type(b, jnp.int32) + 0x8000
    bits = jnp.bitwise_or(jnp.bitwise_and(bi, jnp.int32(-65536)),
                          lax.shift_right_logical(ai, 16))
    ab_ref[...] = lax.bitcast_convert_type(bits, jnp.float32)


def _stage_a(x, w_msg, w_self, w_rel_src, w_rel_dst, brel2):
    return pl.pallas_call(
        _stage_a_body,
        grid=(N // BN,),
        in_specs=[
            pl.BlockSpec((BN, D), lambda i: (i, 0)),
            pl.BlockSpec((D, D), lambda i: (0, 0)),
            pl.BlockSpec((D, D), lambda i: (0, 0)),
            pl.BlockSpec((D, 2), lambda i: (0, 0)),
            pl.BlockSpec((D, 2), lambda i: (0, 0)),
            pl.BlockSpec((1, 2), lambda i: (0, 0)),
        ],
        out_specs=[
            pl.BlockSpec((BN, D), lambda i: (i, 0)),
            pl.BlockSpec((BN, D), lambda i: (i, 0)),
            pl.BlockSpec((BN, 1), lambda i: (i, 0)),
        ],
        out_shape=[
            jax.ShapeDtypeStruct((N, D), jnp.float32),
            jax.ShapeDtypeStruct((N, D), jnp.float32),
            jax.ShapeDtypeStruct((N, 1), jnp.float32),
        ],
    )(x, w_msg, w_self, w_rel_src, w_rel_dst, brel2)


def _stage_b_body(xs_ref, agg_ref, wout_ref, bout_ref, out_ref):
    acc = xs_ref[...] + agg_ref[0] + agg_ref[1]
    h = jnp.maximum(acc, 0.0)
    out_ref[...] = (jnp.dot(h, wout_ref[...], preferred_element_type=jnp.float32)
                    + bout_ref[...])


def _stage_b(xs, agg2, w_out, bout2):
    return pl.pallas_call(
        _stage_b_body,
        grid=(N // BN,),
        in_specs=[
            pl.BlockSpec((BN, D), lambda i: (i, 0)),
            pl.BlockSpec((NC, BN, D), lambda i: (0, i, 0)),
            pl.BlockSpec((D, O), lambda i: (0, 0)),
            pl.BlockSpec((1, O), lambda i: (0, 0)),
        ],
        out_specs=pl.BlockSpec((BN, O), lambda i: (i, 0)),
        out_shape=jax.ShapeDtypeStruct((N, O), jnp.float32),
    )(xs, agg2, w_out, bout2)


def _sc_edge_body(xm_hbm, ab_hbm, edge_hbm, out_hbm,
                  src_v, dst_v, ab_v, rows_v, sidx_v, agg_sh, gsem, ssem):
    cid = lax.axis_index("c")
    sid = lax.axis_index("s")
    wid = cid * NS + sid
    ebase = wid * EPW

    pltpu.sync_copy(edge_hbm.at[pl.ds(ebase, EPW)], src_v.at[pl.ds(0, EPW)])
    pltpu.sync_copy(edge_hbm.at[pl.ds(E + ebase, EPW)], dst_v.at[pl.ds(0, EPW)])
    pltpu.sync_copy(ab_hbm, ab_v)

    zeros16 = jnp.zeros((16,), jnp.float32)
    for r in range(8):
        for c in range(D // 16):
            rows_v[0, r, pl.ds(c * 16, 16)] = zeros16
    rbase = sid * ROWS_PER_TILE

    def zero_loop(i, carry):
        pltpu.sync_copy(rows_v.at[0, pl.ds(0, 8)],
                        agg_sh.at[pl.ds(rbase + i * 8, 8)])
        return carry

    lax.fori_loop(0, ROWS_PER_TILE // 8, zero_loop, 0)

    def compact(g, off):
        sv = src_v[pl.ds(g * CC, CC)]
        dv = dst_v[pl.ds(g * CC, CC)]
        keep = sv <= dv
        plsc.store_compressed(src_v.at[pl.ds(off, CC)], sv, mask=keep)
        plsc.store_compressed(dst_v.at[pl.ds(off, CC)], dv, mask=keep)
        return off + plsc.all_reduce_population_count(keep)[0]

    off = lax.fori_loop(0, NCHC, compact, 0)
    for k in range(PAD // CC):
        src_v[pl.ds(off + k * CC, CC)] = jnp.full((CC,), 1, jnp.int32)
        dst_v[pl.ds(off + k * CC, CC)] = jnp.zeros((CC,), jnp.int32)
    nch2 = lax.max((off + CB - 1) // CB, NBUF)
    plsc.subcore_barrier()

    def gather_start(g, b):
        pltpu.async_copy(xm_hbm.at[src_v.at[pl.ds(g * CB, CB)]],
                         rows_v.at[b], gsem.at[b])

    def gather_wait(g, b):
        pltpu.make_async_copy(xm_hbm.at[src_v.at[pl.ds(g * CB, CB)]],
                              rows_v.at[b], gsem.at[b]).wait()

    def scatter_start(b):
        pltpu.async_copy(rows_v.at[b], agg_sh.at[sidx_v.at[b]], ssem.at[b],
                         add=True)

    def scatter_wait(b):
        pltpu.make_async_copy(rows_v.at[b], agg_sh.at[sidx_v.at[b]],
                              ssem.at[b]).wait()

    for b in range(PREF):
        gather_start(b, b)

    himask = jnp.full((16,), -65536, jnp.int32)

    def slot(g, carry):
        b = lax.rem(g, NBUF)
        gather_wait(g, b)
        for q in range(CB // 16):
            sv = src_v[pl.ds(g * CB + q * 16, 16)]
            dv = dst_v[pl.ds(g * CB + q * 16, 16)]
            g1 = plsc.bitcast(plsc.load_gather(ab_v, [sv]), jnp.int32)
            g2 = plsc.bitcast(plsc.load_gather(ab_v, [dv]), jnp.int32)
            av = plsc.bitcast(lax.shift_left(g1, 16), jnp.float32)
            bv = plsc.bitcast(g2 & himask, jnp.float32)
            p = 1.0 / (1.0 + jnp.exp(-(av + bv)))
            w = jnp.where(sv <= dv, p, 0.0)
            sidx_v[b, pl.ds(q * 16, 16)] = dv
            for j in range(16):
                s = w[j]
                r = q * 16 + j
                for c in range(D // 16):
                    rows_v[b, r, pl.ds(c * 16, 16)] = (
                        rows_v[b, r, pl.ds(c * 16, 16)] * s)
        scatter_start(b)
        gn = g + PREF
        bn = lax.rem(gn, NBUF)

        @pl.when(jnp.logical_and(gn >= NBUF, gn < nch2))
        def _():
            scatter_wait(bn)

        @pl.when(gn < nch2)
        def _():
            gather_start(gn, bn)

        return carry

    lax.fori_loop(0, nch2, slot, 0)
    for b in range(NBUF):
        scatter_wait(b)
    plsc.subcore_barrier()

    pltpu.sync_copy(agg_sh.at[pl.ds(rbase, ROWS_PER_TILE)],
                    out_hbm.at[cid, pl.ds(rbase, ROWS_PER_TILE)])


def _sc_edge(xm, ab_packed, edge_index):
    mesh = plsc.VectorSubcoreMesh(core_axis_name="c", subcore_axis_name="s")
    f = pl.kernel(
        _sc_edge_body,
        out_type=jax.ShapeDtypeStruct((NC, NP, D), jnp.float32),
        mesh=mesh,
        compiler_params=pltpu.CompilerParams(needs_layout_passes=False),
        scratch_types=[
            pltpu.VMEM((EPW + PAD,), jnp.int32),
            pltpu.VMEM((EPW + PAD,), jnp.int32),
            pltpu.VMEM((N,), jnp.float32),
            pltpu.VMEM((NBUF, CB, D), jnp.float32),
            pltpu.VMEM((NBUF, CB), jnp.int32),
            pltpu.VMEM_SHARED((NP, D), jnp.float32),
            pltpu.SemaphoreType.DMA((NBUF,)),
            pltpu.SemaphoreType.DMA((NBUF,)),
        ],
    )
    return f(xm, ab_packed, edge_index)


def kernel(x, edge_index, W_rel_src, W_rel_dst, b_rel, W_msg, W_self, W_out,
           b_out):
    xm, xs, ab = _stage_a(x, W_msg, W_self, W_rel_src, W_rel_dst,
                          b_rel.reshape(1, 2))
    agg2 = _sc_edge(xm, ab.reshape(-1), edge_index.reshape(-1))
    return _stage_b(xs, agg2, W_out, b_out.reshape(1, O))


# speedup vs baseline: 1.2757x; 1.0886x over previous
import jax
import jax.numpy as jnp
from jax import lax
from jax.experimental import pallas as pl
from jax.experimental.pallas import tpu as pltpu
from jax.experimental.pallas import tpu_sc as plsc

N = 10000
E = 320000
D = 128
O = 64

NC = 2
NS = 16
NW = NC * NS
EPW = E // NW
CC = 16
CB = 16
NBUF = 8
PREF = 7
PAD = NBUF * CB
NCHC = EPW // CC
NP = 10112
ROWS_PER_TILE = NP // NS

BN = 2000


def _stage_a_body(x_ref, wmsg_ref, wself_ref, wsrc_ref, wdst_ref, brel_ref,
                  xm_ref, xs_ref, ab_ref):
    xb = x_ref[...]
    xm_ref[...] = jnp.dot(xb, wmsg_ref[...], preferred_element_type=jnp.float32)
    xs_ref[...] = jnp.dot(xb, wself_ref[...], preferred_element_type=jnp.float32)
    ls = (jnp.dot(xb, wsrc_ref[...], preferred_element_type=jnp.float32)
          + brel_ref[...])
    ld = jnp.dot(xb, wdst_ref[...], preferred_element_type=jnp.float32)
    a = ls[:, 1:2] - ls[:, 0:1]
    b = ld[:, 1:2] - ld[:, 0:1]
    ai = lax.bitcast_convert_type(a, jnp.int32) + 0x8000
    bi = lax.bitcast_convert_type(b, jnp.int32) + 0x8000
    bits = jnp.bitwise_or(jnp.bitwise_and(bi, jnp.int32(-65536)),
                          lax.shift_right_logical(ai, 16))
    ab_ref[...] = lax.bitcast_convert_type(bits, jnp.float32)


def _stage_a(x, w_msg, w_self, w_rel_src, w_rel_dst, brel2):
    return pl.pallas_call(
        _stage_a_body,
        grid=(N // BN,),
        in_specs=[
            pl.BlockSpec((BN, D), lambda i: (i, 0)),
            pl.BlockSpec((D, D), lambda i: (0, 0)),
            pl.BlockSpec((D, D), lambda i: (0, 0)),
            pl.BlockSpec((D, 2), lambda i: (0, 0)),
            pl.BlockSpec((D, 2), lambda i: (0, 0)),
            pl.BlockSpec((1, 2), lambda i: (0, 0)),
        ],
        out_specs=[
            pl.BlockSpec((BN, D), lambda i: (i, 0)),
            pl.BlockSpec((BN, D), lambda i: (i, 0)),
            pl.BlockSpec((BN, 1), lambda i: (i, 0)),
        ],
        out_shape=[
            jax.ShapeDtypeStruct((N, D), jnp.float32),
            jax.ShapeDtypeStruct((N, D), jnp.float32),
            jax.ShapeDtypeStruct((N, 1), jnp.float32),
        ],
    )(x, w_msg, w_self, w_rel_src, w_rel_dst, brel2)


def _stage_b_body(xs_ref, agg_ref, wout_ref, bout_ref, out_ref):
    acc = xs_ref[...] + agg_ref[0] + agg_ref[1]
    h = jnp.maximum(acc, 0.0)
    out_ref[...] = (jnp.dot(h, wout_ref[...], preferred_element_type=jnp.float32)
                    + bout_ref[...])


def _stage_b(xs, agg2, w_out, bout2):
    return pl.pallas_call(
        _stage_b_body,
        grid=(N // BN,),
        in_specs=[
            pl.BlockSpec((BN, D), lambda i: (i, 0)),
            pl.BlockSpec((NC, BN, D), lambda i: (0, i, 0)),
            pl.BlockSpec((D, O), lambda i: (0, 0)),
            pl.BlockSpec((1, O), lambda i: (0, 0)),
        ],
        out_specs=pl.BlockSpec((BN, O), lambda i: (i, 0)),
        out_shape=jax.ShapeDtypeStruct((N, O), jnp.float32),
    )(xs, agg2, w_out, bout2)


def _sc_edge_body(xm_hbm, ab_hbm, edge_hbm, out_hbm,
                  src_v, dst_v, ab_v, rows_v, sidx_v, agg_sh, gsem, ssem):
    cid = lax.axis_index("c")
    sid = lax.axis_index("s")
    wid = cid * NS + sid
    ebase = wid * EPW

    pltpu.sync_copy(edge_hbm.at[pl.ds(ebase, EPW)], src_v.at[pl.ds(0, EPW)])
    pltpu.sync_copy(edge_hbm.at[pl.ds(E + ebase, EPW)], dst_v.at[pl.ds(0, EPW)])
    pltpu.sync_copy(ab_hbm, ab_v)

    zeros16 = jnp.zeros((16,), jnp.float32)
    for r in range(8):
        for c in range(D // 16):
            rows_v[0, r, pl.ds(c * 16, 16)] = zeros16
    rbase = sid * ROWS_PER_TILE

    def zero_loop(i, carry):
        pltpu.sync_copy(rows_v.at[0, pl.ds(0, 8)],
                        agg_sh.at[pl.ds(rbase + i * 8, 8)])
        return carry

    lax.fori_loop(0, ROWS_PER_TILE // 8, zero_loop, 0)

    def compact(g, off):
        sv = src_v[pl.ds(g * CC, CC)]
        dv = dst_v[pl.ds(g * CC, CC)]
        keep = sv <= dv
        plsc.store_compressed(src_v.at[pl.ds(off, CC)], sv, mask=keep)
        plsc.store_compressed(dst_v.at[pl.ds(off, CC)], dv, mask=keep)
        return off + plsc.all_reduce_population_count(keep)[0]

    off = lax.fori_loop(0, NCHC, compact, 0)
    for k in range(PAD // CC):
        src_v[pl.ds(off + k * CC, CC)] = jnp.full((CC,), 1, jnp.int32)
        dst_v[pl.ds(off + k * CC, CC)] = jnp.zeros((CC,), jnp.int32)
    nch2 = lax.max((off + CB - 1) // CB, NBUF)
    plsc.subcore_barrier()

    def gather_start(g, b):
        pltpu.async_copy(xm_hbm.at[src_v.at[pl.ds(g * CB, CB)]],
                         rows_v.at[b], gsem.at[b])

    def gather_wait(g, b):
        pltpu.make_async_copy(xm_hbm.at[src_v.at[pl.ds(g * CB, CB)]],
                              rows_v.at[b], gsem.at[b]).wait()

    def scatter_start(b):
        pltpu.async_copy(rows_v.at[b], agg_sh.at[sidx_v.at[b]], ssem.at[b],
                         add=True)

    def scatter_wait(b):
        pltpu.make_async_copy(rows_v.at[b], agg_sh.at[sidx_v.at[b]],
                              ssem.at[b]).wait()

    for b in range(PREF):
        gather_start(b, b)

    himask = jnp.full((16,), -65536, jnp.int32)

    def slot(g, carry):
        b = lax.rem(g, NBUF)
        gather_wait(g, b)
        for q in range(CB // 16):
            sv = src_v[pl.ds(g * CB + q * 16, 16)]
            dv = dst_v[pl.ds(g * CB + q * 16, 16)]
            g1 = plsc.bitcast(plsc.load_gather(ab_v, [sv]), jnp.int32)
            g2 = plsc.bitcast(plsc.load_gather(ab_v, [dv]), jnp.int32)
            av = plsc.bitcast(lax.shift_left(g1, 16), jnp.float32)
            bv = plsc.bitcast(g2 & himask, jnp.float32)
            p = 1.0 / (1.0 + jnp.exp(-(av + bv)))
            w = jnp.where(sv <= dv, p, 0.0)
            sidx_v[b, pl.ds(q * 16, 16)] = dv
            for j in range(16):
                s = w[j]
                r = q * 16 + j
                for c in range(D // 16):
                    rows_v[b, r, pl.ds(c * 16, 16)] = (
                        rows_v[b, r, pl.ds(c * 16, 16)] * s)
        scatter_start(b)
        gn = g + PREF
        bn = lax.rem(gn, NBUF)

        @pl.when(jnp.logical_and(gn >= NBUF, gn < nch2))
        def _():
            scatter_wait(bn)

        @pl.when(gn < nch2)
        def _():
            gather_start(gn, bn)

        return carry

    lax.fori_loop(0, nch2, slot, 0)
    for b in range(NBUF):
        scatter_wait(b)
    plsc.subcore_barrier()

    pltpu.sync_copy(agg_sh.at[pl.ds(rbase, ROWS_PER_TILE)],
                    out_hbm.at[cid, pl.ds(rbase, ROWS_PER_TILE)])


def _sc_edge(xm, ab_packed, edge_index):
    mesh = plsc.VectorSubcoreMesh(core_axis_name="c", subcore_axis_name="s")
    f = pl.kernel(
        _sc_edge_body,
        out_type=jax.ShapeDtypeStruct((NC, NP, D), jnp.float32),
        mesh=mesh,
        compiler_params=pltpu.CompilerParams(needs_layout_passes=False),
        scratch_types=[
            pltpu.VMEM((EPW + PAD,), jnp.int32),
            pltpu.VMEM((EPW + PAD,), jnp.int32),
            pltpu.VMEM((N,), jnp.float32),
            pltpu.VMEM((NBUF, CB, D), jnp.float32),
            pltpu.VMEM((NBUF, CB), jnp.int32),
            pltpu.VMEM_SHARED((NP, D), jnp.float32),
            pltpu.SemaphoreType.DMA((NBUF,)),
            pltpu.SemaphoreType.DMA((NBUF,)),
        ],
    )
    return f(xm, ab_packed, edge_index)


def kernel(x, edge_index, W_rel_src, W_rel_dst, b_rel, W_msg, W_self, W_out,
           b_out):
    xm, xs, ab = _stage_a(x, W_msg, W_self, W_rel_src, W_rel_dst,
                          b_rel.reshape(1, 2))
    agg2 = _sc_edge(xm, ab.reshape(-1), edge_index.reshape(-1))
    return _stage_b(xs, agg2, W_out, b_out.reshape(1, O))


# speedup vs baseline: 1.3591x; 1.0653x over previous
import jax
import jax.numpy as jnp
from jax import lax
from jax.experimental import pallas as pl
from jax.experimental.pallas import tpu as pltpu
from jax.experimental.pallas import tpu_sc as plsc

N = 10000
E = 320000
D = 128
O = 64

NC = 2
NS = 16
NW = NC * NS
EPW = E // NW
CC = 16
CB = 16
NBUF = 8
PREF = 7
PAD = NBUF * CB
NCHC = EPW // CC
NP = 10112
ROWS_PER_TILE = NP // NS

BN = 2000


def _stage_a_body(x_ref, wmsg_ref, wself_ref, wsrc_ref, wdst_ref, brel_ref,
                  xm_ref, xs_ref, ab_ref):
    xb = x_ref[...]
    xm_ref[...] = jnp.dot(xb, wmsg_ref[...], preferred_element_type=jnp.float32)
    xs_ref[...] = jnp.dot(xb, wself_ref[...], preferred_element_type=jnp.float32)
    ls = (jnp.dot(xb, wsrc_ref[...], preferred_element_type=jnp.float32)
          + brel_ref[...])
    ld = jnp.dot(xb, wdst_ref[...], preferred_element_type=jnp.float32)
    a = ls[:, 1:2] - ls[:, 0:1]
    b = ld[:, 1:2] - ld[:, 0:1]
    ai = lax.bitcast_convert_type(a, jnp.int32) + 0x8000
    bi = lax.bitcast_convert_type(b, jnp.int32) + 0x8000
    bits = jnp.bitwise_or(jnp.bitwise_and(bi, jnp.int32(-65536)),
                          lax.shift_right_logical(ai, 16))
    ab_ref[...] = lax.bitcast_convert_type(bits, jnp.float32)


def _stage_a(x, w_msg, w_self, w_rel_src, w_rel_dst, brel2):
    return pl.pallas_call(
        _stage_a_body,
        grid=(N // BN,),
        in_specs=[
            pl.BlockSpec((BN, D), lambda i: (i, 0)),
            pl.BlockSpec((D, D), lambda i: (0, 0)),
            pl.BlockSpec((D, D), lambda i: (0, 0)),
            pl.BlockSpec((D, 2), lambda i: (0, 0)),
            pl.BlockSpec((D, 2), lambda i: (0, 0)),
            pl.BlockSpec((1, 2), lambda i: (0, 0)),
        ],
        out_specs=[
            pl.BlockSpec((BN, D), lambda i: (i, 0)),
            pl.BlockSpec((BN, D), lambda i: (i, 0)),
            pl.BlockSpec((BN, 1), lambda i: (i, 0)),
        ],
        out_shape=[
            jax.ShapeDtypeStruct((N, D), jnp.float32),
            jax.ShapeDtypeStruct((N, D), jnp.float32),
            jax.ShapeDtypeStruct((N, 1), jnp.float32),
        ],
    )(x, w_msg, w_self, w_rel_src, w_rel_dst, brel2)


def _stage_b_body(xs_ref, agg_ref, wout_ref, bout_ref, out_ref):
    acc = xs_ref[...] + agg_ref[0] + agg_ref[1]
    h = jnp.maximum(acc, 0.0)
    out_ref[...] = (jnp.dot(h, wout_ref[...], preferred_element_type=jnp.float32)
                    + bout_ref[...])


def _stage_b(xs, agg2, w_out, bout2):
    return pl.pallas_call(
        _stage_b_body,
        grid=(N // BN,),
        in_specs=[
            pl.BlockSpec((BN, D), lambda i: (i, 0)),
            pl.BlockSpec((NC, BN, D), lambda i: (0, i, 0)),
            pl.BlockSpec((D, O), lambda i: (0, 0)),
            pl.BlockSpec((1, O), lambda i: (0, 0)),
        ],
        out_specs=pl.BlockSpec((BN, O), lambda i: (i, 0)),
        out_shape=jax.ShapeDtypeStruct((N, O), jnp.float32),
    )(xs, agg2, w_out, bout2)


def _sc_edge_body(xm_hbm, ab_hbm, edge_hbm, out_hbm,
                  src_v, dst_v, ab_v, rows_v, sidx_v, agg_sh, gsem, ssem):
    cid = lax.axis_index("c")
    sid = lax.axis_index("s")
    wid = cid * NS + sid
    ebase = wid * EPW

    stage_src = pltpu.make_async_copy(edge_hbm.at[pl.ds(ebase, EPW)],
                                      src_v.at[pl.ds(0, EPW)], gsem.at[0])
    stage_dst = pltpu.make_async_copy(edge_hbm.at[pl.ds(E + ebase, EPW)],
                                      dst_v.at[pl.ds(0, EPW)], gsem.at[1])
    stage_ab = pltpu.make_async_copy(ab_hbm, ab_v, gsem.at[2])
    stage_src.start()
    stage_dst.start()
    stage_ab.start()

    zeros16 = jnp.zeros((16,), jnp.float32)
    for r in range(8):
        for c in range(D // 16):
            rows_v[0, r, pl.ds(c * 16, 16)] = zeros16
    rbase = sid * ROWS_PER_TILE

    def zero_issue(i, carry):
        pltpu.async_copy(rows_v.at[0, pl.ds(0, 8)],
                         agg_sh.at[pl.ds(rbase + i * 8, 8)], ssem.at[0])
        return carry

    lax.fori_loop(0, ROWS_PER_TILE // 8, zero_issue, 0)
    stage_src.wait()
    stage_dst.wait()

    def compact(g, off):
        sv = src_v[pl.ds(g * CC, CC)]
        dv = dst_v[pl.ds(g * CC, CC)]
        keep = sv <= dv
        plsc.store_compressed(src_v.at[pl.ds(off, CC)], sv, mask=keep)
        plsc.store_compressed(dst_v.at[pl.ds(off, CC)], dv, mask=keep)
        return off + plsc.all_reduce_population_count(keep)[0]

    off = lax.fori_loop(0, NCHC, compact, 0)
    for k in range(PAD // CC):
        src_v[pl.ds(off + k * CC, CC)] = jnp.full((CC,), 1, jnp.int32)
        dst_v[pl.ds(off + k * CC, CC)] = jnp.zeros((CC,), jnp.int32)
    nch2 = lax.max((off + CB - 1) // CB, NBUF)
    stage_ab.wait()

    def zero_drain(i, carry):
        pltpu.make_async_copy(rows_v.at[0, pl.ds(0, 8)],
                              agg_sh.at[pl.ds(rbase, 8)], ssem.at[0]).wait()
        return carry

    lax.fori_loop(0, ROWS_PER_TILE // 8, zero_drain, 0)
    plsc.subcore_barrier()

    def gather_start(g, b):
        pltpu.async_copy(xm_hbm.at[src_v.at[pl.ds(g * CB, CB)]],
                         rows_v.at[b], gsem.at[b])

    def gather_wait(g, b):
        pltpu.make_async_copy(xm_hbm.at[src_v.at[pl.ds(g * CB, CB)]],
                              rows_v.at[b], gsem.at[b]).wait()

    def scatter_start(b):
        pltpu.async_copy(rows_v.at[b], agg_sh.at[sidx_v.at[b]], ssem.at[b],
                         add=True)

    def scatter_wait(b):
        pltpu.make_async_copy(rows_v.at[b], agg_sh.at[sidx_v.at[b]],
                              ssem.at[b]).wait()

    for b in range(PREF):
        gather_start(b, b)

    himask = jnp.full((16,), -65536, jnp.int32)

    def slot(g, carry):
        b = lax.rem(g, NBUF)
        gather_wait(g, b)
        for q in range(CB // 16):
            sv = src_v[pl.ds(g * CB + q * 16, 16)]
            dv = dst_v[pl.ds(g * CB + q * 16, 16)]
            g1 = plsc.bitcast(plsc.load_gather(ab_v, [sv]), jnp.int32)
            g2 = plsc.bitcast(plsc.load_gather(ab_v, [dv]), jnp.int32)
            av = plsc.bitcast(lax.shift_left(g1, 16), jnp.float32)
            bv = plsc.bitcast(g2 & himask, jnp.float32)
            p = 1.0 / (1.0 + jnp.exp(-(av + bv)))
            w = jnp.where(sv <= dv, p, 0.0)
            sidx_v[b, pl.ds(q * 16, 16)] = dv
            for j in range(16):
                s = w[j]
                r = q * 16 + j
                for c in range(D // 16):
                    rows_v[b, r, pl.ds(c * 16, 16)] = (
                        rows_v[b, r, pl.ds(c * 16, 16)] * s)
        scatter_start(b)
        gn = g + PREF
        bn = lax.rem(gn, NBUF)

        @pl.when(jnp.logical_and(gn >= NBUF, gn < nch2))
        def _():
            scatter_wait(bn)

        @pl.when(gn < nch2)
        def _():
            gather_start(gn, bn)

        return carry

    lax.fori_loop(0, nch2, slot, 0)
    for b in range(NBUF):
        scatter_wait(b)
    plsc.subcore_barrier()

    pltpu.sync_copy(agg_sh.at[pl.ds(rbase, ROWS_PER_TILE)],
                    out_hbm.at[cid, pl.ds(rbase, ROWS_PER_TILE)])


def _sc_edge(xm, ab_packed, edge_index):
    mesh = plsc.VectorSubcoreMesh(core_axis_name="c", subcore_axis_name="s")
    f = pl.kernel(
        _sc_edge_body,
        out_type=jax.ShapeDtypeStruct((NC, NP, D), jnp.float32),
        mesh=mesh,
        compiler_params=pltpu.CompilerParams(needs_layout_passes=False),
        scratch_types=[
            pltpu.VMEM((EPW + PAD,), jnp.int32),
            pltpu.VMEM((EPW + PAD,), jnp.int32),
            pltpu.VMEM((N,), jnp.float32),
            pltpu.VMEM((NBUF, CB, D), jnp.float32),
            pltpu.VMEM((NBUF, CB), jnp.int32),
            pltpu.VMEM_SHARED((NP, D), jnp.float32),
            pltpu.SemaphoreType.DMA((NBUF,)),
            pltpu.SemaphoreType.DMA((NBUF,)),
        ],
    )
    return f(xm, ab_packed, edge_index)


def kernel(x, edge_index, W_rel_src, W_rel_dst, b_rel, W_msg, W_self, W_out,
           b_out):
    xm, xs, ab = _stage_a(x, W_msg, W_self, W_rel_src, W_rel_dst,
                          b_rel.reshape(1, 2))
    agg2 = _sc_edge(xm, ab.reshape(-1), edge_index.reshape(-1))
    return _stage_b(xs, agg2, W_out, b_out.reshape(1, O))


# speedup vs baseline: 1.3676x; 1.0062x over previous
import jax
import jax.numpy as jnp
from jax import lax
from jax.experimental import pallas as pl
from jax.experimental.pallas import tpu as pltpu
from jax.experimental.pallas import tpu_sc as plsc

N = 10000
E = 320000
D = 128
O = 64

NC = 2
NS = 16
NW = NC * NS
EPW = E // NW
CC = 16
CB = 16
NBUF = 8
PREF = 7
PAD = NBUF * CB
NCHC = EPW // CC
NP = 10112
ROWS_PER_TILE = NP // NS

BN = 2000


def _stage_a_body(x_ref, wmsg_ref, wsrc_ref, wdst_ref, brel_ref,
                  xm_ref, ab_ref):
    xb = x_ref[...]
    xm_ref[...] = jnp.dot(xb, wmsg_ref[...], preferred_element_type=jnp.float32)
    ls = (jnp.dot(xb, wsrc_ref[...], preferred_element_type=jnp.float32)
          + brel_ref[...])
    ld = jnp.dot(xb, wdst_ref[...], preferred_element_type=jnp.float32)
    a = ls[:, 1:2] - ls[:, 0:1]
    b = ld[:, 1:2] - ld[:, 0:1]
    ai = lax.bitcast_convert_type(a, jnp.int32) + 0x8000
    bi = lax.bitcast_convert_type(b, jnp.int32) + 0x8000
    bits = jnp.bitwise_or(jnp.bitwise_and(bi, jnp.int32(-65536)),
                          lax.shift_right_logical(ai, 16))
    ab_ref[...] = lax.bitcast_convert_type(bits, jnp.float32)


def _stage_a(x, w_msg, w_rel_src, w_rel_dst, brel2):
    return pl.pallas_call(
        _stage_a_body,
        grid=(N // BN,),
        in_specs=[
            pl.BlockSpec((BN, D), lambda i: (i, 0)),
            pl.BlockSpec((D, D), lambda i: (0, 0)),
            pl.BlockSpec((D, 2), lambda i: (0, 0)),
            pl.BlockSpec((D, 2), lambda i: (0, 0)),
            pl.BlockSpec((1, 2), lambda i: (0, 0)),
        ],
        out_specs=[
            pl.BlockSpec((BN, D), lambda i: (i, 0)),
            pl.BlockSpec((BN, 1), lambda i: (i, 0)),
        ],
        out_shape=[
            jax.ShapeDtypeStruct((N, D), jnp.float32),
            jax.ShapeDtypeStruct((N, 1), jnp.float32),
        ],
    )(x, w_msg, w_rel_src, w_rel_dst, brel2)


def _stage_b_body(x_ref, wself_ref, agg_ref, wout_ref, bout_ref, out_ref):
    xs = jnp.dot(x_ref[...], wself_ref[...], preferred_element_type=jnp.float32)
    acc = xs + agg_ref[0] + agg_ref[1]
    h = jnp.maximum(acc, 0.0)
    out_ref[...] = (jnp.dot(h, wout_ref[...], preferred_element_type=jnp.float32)
                    + bout_ref[...])


def _stage_b(x, w_self, agg2, w_out, bout2):
    return pl.pallas_call(
        _stage_b_body,
        grid=(N // BN,),
        in_specs=[
            pl.BlockSpec((BN, D), lambda i: (i, 0)),
            pl.BlockSpec((D, D), lambda i: (0, 0)),
            pl.BlockSpec((NC, BN, D), lambda i: (0, i, 0)),
            pl.BlockSpec((D, O), lambda i: (0, 0)),
            pl.BlockSpec((1, O), lambda i: (0, 0)),
        ],
        out_specs=pl.BlockSpec((BN, O), lambda i: (i, 0)),
        out_shape=jax.ShapeDtypeStruct((N, O), jnp.float32),
    )(x, w_self, agg2, w_out, bout2)


def _sc_edge_body(xm_hbm, ab_hbm, edge_hbm, out_hbm,
                  src_v, dst_v, ab_v, rows_v, sidx_v, agg_sh, gsem, ssem):
    cid = lax.axis_index("c")
    sid = lax.axis_index("s")
    wid = cid * NS + sid
    ebase = wid * EPW

    stage_src = pltpu.make_async_copy(edge_hbm.at[pl.ds(ebase, EPW)],
                                      src_v.at[pl.ds(0, EPW)], gsem.at[0])
    stage_dst = pltpu.make_async_copy(edge_hbm.at[pl.ds(E + ebase, EPW)],
                                      dst_v.at[pl.ds(0, EPW)], gsem.at[1])
    stage_ab = pltpu.make_async_copy(ab_hbm, ab_v, gsem.at[2])
    stage_src.start()
    stage_dst.start()
    stage_ab.start()

    zeros16 = jnp.zeros((16,), jnp.float32)
    for r in range(8):
        for c in range(D // 16):
            rows_v[0, r, pl.ds(c * 16, 16)] = zeros16
    rbase = sid * ROWS_PER_TILE

    def zero_issue(i, carry):
        pltpu.async_copy(rows_v.at[0, pl.ds(0, 8)],
                         agg_sh.at[pl.ds(rbase + i * 8, 8)], ssem.at[0])
        return carry

    lax.fori_loop(0, ROWS_PER_TILE // 8, zero_issue, 0)
    stage_src.wait()
    stage_dst.wait()

    def compact(g, off):
        sv = src_v[pl.ds(g * CC, CC)]
        dv = dst_v[pl.ds(g * CC, CC)]
        keep = sv <= dv
        plsc.store_compressed(src_v.at[pl.ds(off, CC)], sv, mask=keep)
        plsc.store_compressed(dst_v.at[pl.ds(off, CC)], dv, mask=keep)
        return off + plsc.all_reduce_population_count(keep)[0]

    off = lax.fori_loop(0, NCHC, compact, 0)
    for k in range(PAD // CC):
        src_v[pl.ds(off + k * CC, CC)] = jnp.full((CC,), 1, jnp.int32)
        dst_v[pl.ds(off + k * CC, CC)] = jnp.zeros((CC,), jnp.int32)
    nch2 = lax.max((off + CB - 1) // CB, NBUF)
    stage_ab.wait()

    def zero_drain(i, carry):
        pltpu.make_async_copy(rows_v.at[0, pl.ds(0, 8)],
                              agg_sh.at[pl.ds(rbase, 8)], ssem.at[0]).wait()
        return carry

    lax.fori_loop(0, ROWS_PER_TILE // 8, zero_drain, 0)
    plsc.subcore_barrier()

    def gather_start(g, b):
        pltpu.async_copy(xm_hbm.at[src_v.at[pl.ds(g * CB, CB)]],
                         rows_v.at[b], gsem.at[b])

    def gather_wait(g, b):
        pltpu.make_async_copy(xm_hbm.at[src_v.at[pl.ds(g * CB, CB)]],
                              rows_v.at[b], gsem.at[b]).wait()

    def scatter_start(b):
        pltpu.async_copy(rows_v.at[b], agg_sh.at[sidx_v.at[b]], ssem.at[b],
                         add=True)

    def scatter_wait(b):
        pltpu.make_async_copy(rows_v.at[b], agg_sh.at[sidx_v.at[b]],
                              ssem.at[b]).wait()

    for b in range(PREF):
        gather_start(b, b)

    himask = jnp.full((16,), -65536, jnp.int32)

    def slot(g, carry):
        b = lax.rem(g, NBUF)
        gather_wait(g, b)
        for q in range(CB // 16):
            sv = src_v[pl.ds(g * CB + q * 16, 16)]
            dv = dst_v[pl.ds(g * CB + q * 16, 16)]
            g1 = plsc.bitcast(plsc.load_gather(ab_v, [sv]), jnp.int32)
            g2 = plsc.bitcast(plsc.load_gather(ab_v, [dv]), jnp.int32)
            av = plsc.bitcast(lax.shift_left(g1, 16), jnp.float32)
            bv = plsc.bitcast(g2 & himask, jnp.float32)
            p = 1.0 / (1.0 + jnp.exp(-(av + bv)))
            w = jnp.where(sv <= dv, p, 0.0)
            sidx_v[b, pl.ds(q * 16, 16)] = dv
            for j in range(16):
                s = w[j]
                r = q * 16 + j
                for c in range(D // 16):
                    rows_v[b, r, pl.ds(c * 16, 16)] = (
                        rows_v[b, r, pl.ds(c * 16, 16)] * s)
        scatter_start(b)
        gn = g + PREF
        bn = lax.rem(gn, NBUF)

        @pl.when(jnp.logical_and(gn >= NBUF, gn < nch2))
        def _():
            scatter_wait(bn)

        @pl.when(gn < nch2)
        def _():
            gather_start(gn, bn)

        return carry

    lax.fori_loop(0, nch2, slot, 0)
    for b in range(NBUF):
        scatter_wait(b)
    plsc.subcore_barrier()

    pltpu.sync_copy(agg_sh.at[pl.ds(rbase, ROWS_PER_TILE)],
                    out_hbm.at[cid, pl.ds(rbase, ROWS_PER_TILE)])


def _sc_edge(xm, ab_packed, edge_index):
    mesh = plsc.VectorSubcoreMesh(core_axis_name="c", subcore_axis_name="s")
    f = pl.kernel(
        _sc_edge_body,
        out_type=jax.ShapeDtypeStruct((NC, NP, D), jnp.float32),
        mesh=mesh,
        compiler_params=pltpu.CompilerParams(needs_layout_passes=False),
        scratch_types=[
            pltpu.VMEM((EPW + PAD,), jnp.int32),
            pltpu.VMEM((EPW + PAD,), jnp.int32),
            pltpu.VMEM((N,), jnp.float32),
            pltpu.VMEM((NBUF, CB, D), jnp.float32),
            pltpu.VMEM((NBUF, CB), jnp.int32),
            pltpu.VMEM_SHARED((NP, D), jnp.float32),
            pltpu.SemaphoreType.DMA((NBUF,)),
            pltpu.SemaphoreType.DMA((NBUF,)),
        ],
    )
    return f(xm, ab_packed, edge_index)


def kernel(x, edge_index, W_rel_src, W_rel_dst, b_rel, W_msg, W_self, W_out,
           b_out):
    xm, ab = _stage_a(x, W_msg, W_rel_src, W_rel_dst, b_rel.reshape(1, 2))
    agg2 = _sc_edge(xm, ab.reshape(-1), edge_index.reshape(-1))
    return _stage_b(x, W_self, agg2, W_out, b_out.reshape(1, O))
